# Initial kernel scaffold; baseline (speedup 1.0000x reference)
import jax
import jax.numpy as jnp
from jax.experimental import pallas as pl


def kernel(x, edge_index, batch, W1, b1, Ws1, bs1, W2, b2, Ws2, bs2, W3, b3, Ws3, bs3, Wc1, bc1, Wc2, bc2, Wc3, bc3):
    raise NotImplementedError("write your pallas kernel here")


# speedup vs baseline: 1.0003x; 1.0003x over previous
import functools
import math

import jax
import jax.numpy as jnp
from jax.experimental import pallas as pl
from jax.experimental.pallas import tpu as pltpu

N = 10000
E = 320000
F_IN = 128
H = 128
C = 10
RATIO = 0.5
K1 = math.ceil(RATIO * N)
K2 = math.ceil(RATIO * K1)
K3 = math.ceil(RATIO * K2)


def _mlp_body(z_ref, wc1_ref, bc1_ref, wc2_ref, bc2_ref, wc3_ref, bc3_ref, out_ref):
    z = z_ref[...]
    z = jax.nn.relu(z @ wc1_ref[...] + bc1_ref[...])
    z = jax.nn.relu(z @ wc2_ref[...] + bc2_ref[...])
    z = z @ wc3_ref[...] + bc3_ref[...]
    out_ref[...] = jax.nn.log_softmax(z, axis=-1)


def _mlp_head(z, Wc1, bc1, Wc2, bc2, Wc3, bc3):
    Wc3p = jnp.zeros((H // 2, 128), jnp.float32).at[:, :C].set(Wc3)
    bc3p = jnp.full((128,), -1e30, jnp.float32).at[:C].set(bc3)
    out = pl.pallas_call(
        _mlp_body,
        out_shape=jax.ShapeDtypeStruct((1, 128), jnp.float32),
    )(z, Wc1, bc1[None, :], Wc2, bc2[None, :], Wc3p, bc3p[None, :])
    return out[:, :C]


def _gcn(h, W, b, src, dst, nm, em):
    hw = h @ W
    deg = jnp.zeros((N,), jnp.float32).at[dst].add(em) + nm
    dis = jax.lax.rsqrt(jnp.maximum(deg, 1.0)) * (deg > 0).astype(jnp.float32)
    coef = dis[src] * dis[dst] * em
    out = jnp.zeros_like(hw).at[dst].add(hw[src] * coef[:, None])
    out = out + hw * (dis * dis * nm)[:, None]
    return (out + b) * nm[:, None]


def _sag_pool(h, src, dst, nm, em, Ws, bs, k):
    score = _gcn(h, Ws, bs, src, dst, nm, em)[:, 0]
    masked = jnp.where(nm > 0, score, -jnp.inf)
    _, idx = jax.lax.top_k(masked, k)
    new_nm = jnp.zeros((N,), jnp.float32).at[idx].set(1.0)
    h2 = h * jnp.tanh(score)[:, None] * new_nm[:, None]
    new_em = em * new_nm[src] * new_nm[dst]
    return h2, new_nm, new_em


def _rbf(h, nm, k):
    mu = (h * nm[:, None]).sum(axis=0) / k
    var = (((h - mu) ** 2) * nm[:, None]).sum(axis=0) / (k - 1)
    out = 1.0 / (jnp.sqrt(2.0 * jnp.pi * var) + 1e-6) * jnp.exp(-0.5 * ((h - mu) ** 2) / var)
    return out * nm[:, None]


def _readout(h, nm, k):
    mx = jnp.max(jnp.where(nm[:, None] > 0, h, -jnp.inf), axis=0)
    mn = (h * nm[:, None]).sum(axis=0) / k
    return jnp.concatenate([mx, mn])[None, :]


def kernel(x, edge_index, batch, W1, b1, Ws1, bs1, W2, b2, Ws2, bs2, W3, b3, Ws3, bs3, Wc1, bc1, Wc2, bc2, Wc3, bc3):
    src = edge_index[0]
    dst = edge_index[1]
    nm = jnp.ones((N,), jnp.float32)
    em = jnp.ones((E,), jnp.float32)
    h = jax.nn.relu(_gcn(x, W1, b1, src, dst, nm, em))
    h, nm, em = _sag_pool(h, src, dst, nm, em, Ws1, bs1, K1)
    h = _rbf(h, nm, K1)
    r1 = _readout(h, nm, K1)
    h = jax.nn.relu(_gcn(h, W2, b2, src, dst, nm, em))
    h, nm, em = _sag_pool(h, src, dst, nm, em, Ws2, bs2, K2)
    h = _rbf(h, nm, K2)
    r2 = _readout(h, nm, K2)
    h = jax.nn.relu(_gcn(h, W3, b3, src, dst, nm, em))
    h, nm, em = _sag_pool(h, src, dst, nm, em, Ws3, bs3, K3)
    h = _rbf(h, nm, K3)
    r3 = _readout(h, nm, K3)
    z = r1 + r2 + r3
    return _mlp_head(z, Wc1, bc1, Wc2, bc2, Wc3, bc3)


# speedup vs baseline: 6.1341x; 6.1324x over previous
import functools
import math

import jax
import jax.numpy as jnp
from jax.experimental import pallas as pl
from jax.experimental.pallas import tpu as pltpu
from jax.experimental.pallas import tpu_sc as plsc

N = 10000
E = 320000
F_IN = 128
H = 128
C = 10
RATIO = 0.5
K1 = math.ceil(RATIO * N)
K2 = math.ceil(RATIO * K1)
K3 = math.ceil(RATIO * K2)

NPAD = 10240
EPAD = 327680
CH = 128
NSUB = 16
NCORE = 2
EPT = EPAD // (NSUB * NCORE)
NCHUNK = EPT // CH
RPT = NPAD // NSUB


def _make_edge_agg(D):
    mesh = plsc.VectorSubcoreMesh(core_axis_name="c", subcore_axis_name="s",
                                  num_cores=NCORE)

    @functools.partial(
        pl.kernel, mesh=mesh,
        out_type=jax.ShapeDtypeStruct((NCORE * NPAD, D), jnp.float32),
        scratch_types=[
            pltpu.VMEM((CH,), jnp.int32),
            pltpu.VMEM((CH,), jnp.int32),
            pltpu.VMEM((CH, D), jnp.float32),
            pltpu.VMEM_SHARED((NPAD, D), jnp.float32),
        ],
    )
    def k(vals_hbm, src_hbm, dst_hbm, zero_hbm, out_hbm, idx_s, idx_d, rows, acc):
        c = jax.lax.axis_index("c")
        s = jax.lax.axis_index("s")
        wid = c * NSUB + s
        pltpu.sync_copy(zero_hbm, acc.at[pl.ds(s * RPT, RPT)])
        plsc.subcore_barrier()

        def body(i, carry):
            base = pl.multiple_of(wid * EPT + i * CH, CH)
            pltpu.sync_copy(src_hbm.at[pl.ds(base, CH)], idx_s)
            pltpu.sync_copy(dst_hbm.at[pl.ds(base, CH)], idx_d)
            pltpu.sync_copy(vals_hbm.at[idx_s], rows)
            pltpu.sync_copy(rows, acc.at[idx_d], add=True)
            return carry

        jax.lax.fori_loop(0, NCHUNK, body, 0)
        plsc.subcore_barrier()
        pltpu.sync_copy(acc.at[pl.ds(s * RPT, RPT)],
                        out_hbm.at[pl.ds(c * NPAD + s * RPT, RPT)])

    return k


_edge_agg_128 = _make_edge_agg(128)


def _agg_rows(hwp_pad, srcp, dstp, zero128):
    out = _edge_agg_128(hwp_pad, srcp, dstp, zero128)
    return out[:N] + out[NPAD:NPAD + N]


def _agg_scalar(v, srcp, dstp, zero128):
    vp = jnp.zeros((NPAD, 128), jnp.float32).at[:N, 0].set(v)
    out = _edge_agg_128(vp, srcp, dstp, zero128)
    return out[:N, 0] + out[NPAD:NPAD + N, 0]


def _mlp_body(z_ref, wc1_ref, bc1_ref, wc2_ref, bc2_ref, wc3_ref, bc3_ref, out_ref):
    z = z_ref[...]
    z = jax.nn.relu(z @ wc1_ref[...] + bc1_ref[...])
    z = jax.nn.relu(z @ wc2_ref[...] + bc2_ref[...])
    z = z @ wc3_ref[...] + bc3_ref[...]
    out_ref[...] = jax.nn.log_softmax(z, axis=-1)


def _mlp_head(z, Wc1, bc1, Wc2, bc2, Wc3, bc3):
    Wc3p = jnp.zeros((H // 2, 128), jnp.float32).at[:, :C].set(Wc3)
    bc3p = jnp.full((128,), -1e30, jnp.float32).at[:C].set(bc3)
    out = pl.pallas_call(
        _mlp_body,
        out_shape=jax.ShapeDtypeStruct((1, 128), jnp.float32),
    )(z, Wc1, bc1[None, :], Wc2, bc2[None, :], Wc3p, bc3p[None, :])
    return out[:, :C]


def _rbf(h, nm, k):
    mu = (h * nm[:, None]).sum(axis=0) / k
    var = (((h - mu) ** 2) * nm[:, None]).sum(axis=0) / (k - 1)
    out = 1.0 / (jnp.sqrt(2.0 * jnp.pi * var) + 1e-6) * jnp.exp(-0.5 * ((h - mu) ** 2) / var)
    return out * nm[:, None]


def _readout(h, nm, k):
    mx = jnp.max(jnp.where(nm[:, None] > 0, h, -jnp.inf), axis=0)
    mn = (h * nm[:, None]).sum(axis=0) / k
    return jnp.concatenate([mx, mn])[None, :]


def _layer(h, W, b, Ws, bs, nm, srcp, dstp, zero128, k):
    degagg = _agg_scalar(nm, srcp, dstp, zero128)
    deg = nm * (degagg + 1.0)
    dis = jax.lax.rsqrt(jnp.maximum(deg, 1.0)) * (deg > 0).astype(jnp.float32)
    d2n = dis * dis * nm
    hw = h @ W
    hwp = jnp.zeros((NPAD, H), jnp.float32).at[:N].set(hw * dis[:, None])
    agg = _agg_rows(hwp, srcp, dstp, zero128)
    h1 = jax.nn.relu((agg * dis[:, None] + hw * d2n[:, None] + b) * nm[:, None])
    hs = (h1 @ Ws)[:, 0]
    sagg = _agg_scalar(hs * dis, srcp, dstp, zero128)
    score = (sagg * dis + hs * d2n + bs[0]) * nm
    masked = jnp.where(nm > 0, score, -jnp.inf)
    _, idx = jax.lax.top_k(masked, k)
    new_nm = jnp.zeros((N,), jnp.float32).at[idx].set(1.0)
    h2 = h1 * jnp.tanh(score)[:, None] * new_nm[:, None]
    return h2, new_nm


def kernel(x, edge_index, batch, W1, b1, Ws1, bs1, W2, b2, Ws2, bs2, W3, b3, Ws3, bs3, Wc1, bc1, Wc2, bc2, Wc3, bc3):
    pad = jnp.full((EPAD - E,), NPAD - 1, jnp.int32)
    srcp = jnp.concatenate([edge_index[0], pad])
    dstp = jnp.concatenate([edge_index[1], pad])
    zero128 = jnp.zeros((RPT, 128), jnp.float32)
    nm = jnp.ones((N,), jnp.float32)

    h, nm = _layer(x, W1, b1, Ws1, bs1, nm, srcp, dstp, zero128, K1)
    h = _rbf(h, nm, K1)
    r1 = _readout(h, nm, K1)
    h, nm = _layer(h, W2, b2, Ws2, bs2, nm, srcp, dstp, zero128, K2)
    h = _rbf(h, nm, K2)
    r2 = _readout(h, nm, K2)
    h, nm = _layer(h, W3, b3, Ws3, bs3, nm, srcp, dstp, zero128, K3)
    h = _rbf(h, nm, K3)
    r3 = _readout(h, nm, K3)
    z = r1 + r2 + r3
    return _mlp_head(z, Wc1, bc1, Wc2, bc2, Wc3, bc3)


# speedup vs baseline: 7.5799x; 1.2357x over previous
import functools
import math

import jax
import jax.numpy as jnp
from jax.experimental import pallas as pl
from jax.experimental.pallas import tpu as pltpu
from jax.experimental.pallas import tpu_sc as plsc

N = 10000
E = 320000
F_IN = 128
H = 128
C = 10
RATIO = 0.5
K1 = math.ceil(RATIO * N)
K2 = math.ceil(RATIO * K1)
K3 = math.ceil(RATIO * K2)

NPAD = 10240
EPAD = 327680
CH = 128
NSUB = 16
NCORE = 2
EPT = EPAD // (NSUB * NCORE)
NCHUNK = EPT // CH
RPT = NPAD // NSUB


NBUF = 2


def _make_edge_agg(D):
    mesh = plsc.VectorSubcoreMesh(core_axis_name="c", subcore_axis_name="s",
                                  num_cores=NCORE)

    @functools.partial(
        pl.kernel, mesh=mesh,
        out_type=jax.ShapeDtypeStruct((NCORE * NPAD, D), jnp.float32),
        scratch_types=[
            pltpu.VMEM((NBUF, CH), jnp.int32),
            pltpu.VMEM((NBUF, CH), jnp.int32),
            pltpu.VMEM((NBUF, CH, D), jnp.float32),
            pltpu.VMEM_SHARED((NPAD, D), jnp.float32),
        ] + [pltpu.SemaphoreType.DMA] * NBUF,
    )
    def k(vals_hbm, src_hbm, dst_hbm, zero_hbm, out_hbm, idx_s, idx_d, rows,
          acc, *sems):
        c = jax.lax.axis_index("c")
        s = jax.lax.axis_index("s")
        wid = c * NSUB + s

        def issue(i, b):
            base = pl.multiple_of(wid * EPT + i * CH, CH)
            pltpu.sync_copy(src_hbm.at[pl.ds(base, CH)], idx_s.at[b])
            pltpu.sync_copy(dst_hbm.at[pl.ds(base, CH)], idx_d.at[b])
            pltpu.async_copy(vals_hbm.at[idx_s.at[b]], rows.at[b], sems[b])

        def consume(i, b):
            pltpu.make_async_copy(vals_hbm.at[pl.ds(0, CH)], rows.at[b],
                                  sems[b]).wait()
            pltpu.sync_copy(rows.at[b], acc.at[idx_d.at[b]], add=True)

        for b in range(NBUF):
            issue(b, b)
        pltpu.sync_copy(zero_hbm, acc.at[pl.ds(s * RPT, RPT)])
        plsc.subcore_barrier()

        def body(g, carry):
            for b in range(NBUF):
                i = g * NBUF + b
                consume(i, b)
                issue(i + NBUF, b)
            return carry

        jax.lax.fori_loop(0, NCHUNK // NBUF - 1, body, 0)
        for b in range(NBUF):
            consume(NCHUNK - NBUF + b, b)
        plsc.subcore_barrier()
        pltpu.sync_copy(acc.at[pl.ds(s * RPT, RPT)],
                        out_hbm.at[pl.ds(c * NPAD + s * RPT, RPT)])

    return k


_edge_agg_128 = _make_edge_agg(128)


def _agg_rows(hwp_pad, srcp, dstp, zero128):
    out = _edge_agg_128(hwp_pad, srcp, dstp, zero128)
    return out[:N] + out[NPAD:NPAD + N]


def _agg_scalar(v, srcp, dstp, zero128):
    vp = jnp.zeros((NPAD, 128), jnp.float32).at[:N, 0].set(v)
    out = _edge_agg_128(vp, srcp, dstp, zero128)
    return out[:N, 0] + out[NPAD:NPAD + N, 0]


def _mlp_body(z_ref, wc1_ref, bc1_ref, wc2_ref, bc2_ref, wc3_ref, bc3_ref, out_ref):
    z = z_ref[...]
    z = jax.nn.relu(z @ wc1_ref[...] + bc1_ref[...])
    z = jax.nn.relu(z @ wc2_ref[...] + bc2_ref[...])
    z = z @ wc3_ref[...] + bc3_ref[...]
    out_ref[...] = jax.nn.log_softmax(z, axis=-1)


def _mlp_head(z, Wc1, bc1, Wc2, bc2, Wc3, bc3):
    Wc3p = jnp.zeros((H // 2, 128), jnp.float32).at[:, :C].set(Wc3)
    bc3p = jnp.full((128,), -1e30, jnp.float32).at[:C].set(bc3)
    out = pl.pallas_call(
        _mlp_body,
        out_shape=jax.ShapeDtypeStruct((1, 128), jnp.float32),
    )(z, Wc1, bc1[None, :], Wc2, bc2[None, :], Wc3p, bc3p[None, :])
    return out[:, :C]


def _rbf(h, nm, k):
    mu = (h * nm[:, None]).sum(axis=0) / k
    var = (((h - mu) ** 2) * nm[:, None]).sum(axis=0) / (k - 1)
    out = 1.0 / (jnp.sqrt(2.0 * jnp.pi * var) + 1e-6) * jnp.exp(-0.5 * ((h - mu) ** 2) / var)
    return out * nm[:, None]


def _readout(h, nm, k):
    mx = jnp.max(jnp.where(nm[:, None] > 0, h, -jnp.inf), axis=0)
    mn = (h * nm[:, None]).sum(axis=0) / k
    return jnp.concatenate([mx, mn])[None, :]


def _layer(h, W, b, Ws, bs, nm, srcp, dstp, zero128, k):
    degagg = _agg_scalar(nm, srcp, dstp, zero128)
    deg = nm * (degagg + 1.0)
    dis = jax.lax.rsqrt(jnp.maximum(deg, 1.0)) * (deg > 0).astype(jnp.float32)
    d2n = dis * dis * nm
    hw = h @ W
    hwp = jnp.zeros((NPAD, H), jnp.float32).at[:N].set(hw * dis[:, None])
    agg = _agg_rows(hwp, srcp, dstp, zero128)
    h1 = jax.nn.relu((agg * dis[:, None] + hw * d2n[:, None] + b) * nm[:, None])
    hs = (h1 @ Ws)[:, 0]
    sagg = _agg_scalar(hs * dis, srcp, dstp, zero128)
    score = (sagg * dis + hs * d2n + bs[0]) * nm
    masked = jnp.where(nm > 0, score, -jnp.inf)
    _, idx = jax.lax.top_k(masked, k)
    new_nm = jnp.zeros((N,), jnp.float32).at[idx].set(1.0)
    h2 = h1 * jnp.tanh(score)[:, None] * new_nm[:, None]
    return h2, new_nm


def kernel(x, edge_index, batch, W1, b1, Ws1, bs1, W2, b2, Ws2, bs2, W3, b3, Ws3, bs3, Wc1, bc1, Wc2, bc2, Wc3, bc3):
    pad = jnp.full((EPAD - E,), NPAD - 1, jnp.int32)
    srcp = jnp.concatenate([edge_index[0], pad])
    dstp = jnp.concatenate([edge_index[1], pad])
    zero128 = jnp.zeros((RPT, 128), jnp.float32)
    nm = jnp.ones((N,), jnp.float32)

    h, nm = _layer(x, W1, b1, Ws1, bs1, nm, srcp, dstp, zero128, K1)
    h = _rbf(h, nm, K1)
    r1 = _readout(h, nm, K1)
    h, nm = _layer(h, W2, b2, Ws2, bs2, nm, srcp, dstp, zero128, K2)
    h = _rbf(h, nm, K2)
    r2 = _readout(h, nm, K2)
    h, nm = _layer(h, W3, b3, Ws3, bs3, nm, srcp, dstp, zero128, K3)
    h = _rbf(h, nm, K3)
    r3 = _readout(h, nm, K3)
    z = r1 + r2 + r3
    return _mlp_head(z, Wc1, bc1, Wc2, bc2, Wc3, bc3)


# speedup vs baseline: 21.1819x; 2.7945x over previous
import functools
import math

import jax
import jax.numpy as jnp
from jax.experimental import pallas as pl
from jax.experimental.pallas import tpu as pltpu
from jax.experimental.pallas import tpu_sc as plsc

N = 10000
E = 320000
F_IN = 128
H = 128
C = 10
RATIO = 0.5
K1 = math.ceil(RATIO * N)
K2 = math.ceil(RATIO * K1)
K3 = math.ceil(RATIO * K2)

NPAD = 10240
EPAD = 327680
CH = 128
NSUB = 16
NCORE = 2
EPT = EPAD // (NSUB * NCORE)
NCHUNK = EPT // CH
RPT = NPAD // NSUB


NBUF = 2


def _make_edge_agg(D):
    mesh = plsc.VectorSubcoreMesh(core_axis_name="c", subcore_axis_name="s",
                                  num_cores=NCORE)

    @functools.partial(
        pl.kernel, mesh=mesh,
        out_type=jax.ShapeDtypeStruct((NCORE * NPAD, D), jnp.float32),
        scratch_types=[
            pltpu.VMEM((NBUF, CH), jnp.int32),
            pltpu.VMEM((NBUF, CH), jnp.int32),
            pltpu.VMEM((NBUF, CH, D), jnp.float32),
            pltpu.VMEM_SHARED((NPAD, D), jnp.float32),
        ] + [pltpu.SemaphoreType.DMA] * NBUF,
    )
    def k(vals_hbm, src_hbm, dst_hbm, zero_hbm, out_hbm, idx_s, idx_d, rows,
          acc, *sems):
        c = jax.lax.axis_index("c")
        s = jax.lax.axis_index("s")
        wid = c * NSUB + s

        def issue(i, b):
            base = pl.multiple_of(wid * EPT + i * CH, CH)
            pltpu.sync_copy(src_hbm.at[pl.ds(base, CH)], idx_s.at[b])
            pltpu.sync_copy(dst_hbm.at[pl.ds(base, CH)], idx_d.at[b])
            pltpu.async_copy(vals_hbm.at[idx_s.at[b]], rows.at[b], sems[b])

        def consume(i, b):
            pltpu.make_async_copy(vals_hbm.at[pl.ds(0, CH)], rows.at[b],
                                  sems[b]).wait()
            pltpu.sync_copy(rows.at[b], acc.at[idx_d.at[b]], add=True)

        for b in range(NBUF):
            issue(b, b)
        pltpu.sync_copy(zero_hbm, acc.at[pl.ds(s * RPT, RPT)])
        plsc.subcore_barrier()

        def body(g, carry):
            for b in range(NBUF):
                i = g * NBUF + b
                consume(i, b)
                issue(i + NBUF, b)
            return carry

        jax.lax.fori_loop(0, NCHUNK // NBUF - 1, body, 0)
        for b in range(NBUF):
            consume(NCHUNK - NBUF + b, b)
        plsc.subcore_barrier()
        pltpu.sync_copy(acc.at[pl.ds(s * RPT, RPT)],
                        out_hbm.at[pl.ds(c * NPAD + s * RPT, RPT)])

    return k


_edge_agg_128 = _make_edge_agg(128)


def _agg_rows(hwp_pad, srcp, dstp, zero128):
    out = _edge_agg_128(hwp_pad, srcp, dstp, zero128)
    return out[:N] + out[NPAD:NPAD + N]


def _make_scalar_agg():
    mesh = plsc.VectorSubcoreMesh(core_axis_name="c", subcore_axis_name="s",
                                  num_cores=NCORE)
    NW = NCORE * NSUB

    @functools.partial(
        pl.kernel, mesh=mesh,
        out_type=jax.ShapeDtypeStruct((NW, NPAD), jnp.float32),
        compiler_params=pltpu.CompilerParams(needs_layout_passes=False),
        scratch_types=[
            pltpu.VMEM((NPAD,), jnp.float32),
            pltpu.VMEM((NPAD,), jnp.float32),
            pltpu.VMEM((EPT,), jnp.int32),
            pltpu.VMEM((EPT,), jnp.int32),
        ],
    )
    def k(vals_hbm, src_hbm, dst_hbm, zero_hbm, out_hbm, vals_l, acc_l,
          idx_s, idx_d):
        c = jax.lax.axis_index("c")
        s = jax.lax.axis_index("s")
        wid = c * NSUB + s
        ebase = pl.multiple_of(wid * EPT, CH)
        pltpu.sync_copy(vals_hbm, vals_l)
        pltpu.sync_copy(src_hbm.at[pl.ds(ebase, EPT)], idx_s)
        pltpu.sync_copy(dst_hbm.at[pl.ds(ebase, EPT)], idx_d)
        pltpu.sync_copy(zero_hbm, acc_l)

        def body(i, carry):
            sv = idx_s[pl.ds(i * 16, 16)]
            dv = idx_d[pl.ds(i * 16, 16)]
            v = plsc.load_gather(vals_l, [sv])
            plsc.addupdate_scatter(acc_l, [dv], v)
            return carry

        jax.lax.fori_loop(0, EPT // 16, body, 0)
        pltpu.sync_copy(acc_l, out_hbm.at[wid])

    return k


_scalar_agg = _make_scalar_agg()


def _agg_scalar(v, srcp, dstp, zero128):
    vp = jnp.zeros((NPAD,), jnp.float32).at[:N].set(v)
    out = _scalar_agg(vp, srcp, dstp, jnp.zeros((NPAD,), jnp.float32))
    return out.sum(axis=0)[:N]


def _mlp_body(z_ref, wc1_ref, bc1_ref, wc2_ref, bc2_ref, wc3_ref, bc3_ref, out_ref):
    z = z_ref[...]
    z = jax.nn.relu(z @ wc1_ref[...] + bc1_ref[...])
    z = jax.nn.relu(z @ wc2_ref[...] + bc2_ref[...])
    z = z @ wc3_ref[...] + bc3_ref[...]
    out_ref[...] = jax.nn.log_softmax(z, axis=-1)


def _mlp_head(z, Wc1, bc1, Wc2, bc2, Wc3, bc3):
    Wc3p = jnp.zeros((H // 2, 128), jnp.float32).at[:, :C].set(Wc3)
    bc3p = jnp.full((128,), -1e30, jnp.float32).at[:C].set(bc3)
    out = pl.pallas_call(
        _mlp_body,
        out_shape=jax.ShapeDtypeStruct((1, 128), jnp.float32),
    )(z, Wc1, bc1[None, :], Wc2, bc2[None, :], Wc3p, bc3p[None, :])
    return out[:, :C]


def _rbf(h, nm, k):
    mu = (h * nm[:, None]).sum(axis=0) / k
    var = (((h - mu) ** 2) * nm[:, None]).sum(axis=0) / (k - 1)
    out = 1.0 / (jnp.sqrt(2.0 * jnp.pi * var) + 1e-6) * jnp.exp(-0.5 * ((h - mu) ** 2) / var)
    return out * nm[:, None]


def _readout(h, nm, k):
    mx = jnp.max(jnp.where(nm[:, None] > 0, h, -jnp.inf), axis=0)
    mn = (h * nm[:, None]).sum(axis=0) / k
    return jnp.concatenate([mx, mn])[None, :]


def _layer(h, W, b, Ws, bs, nm, srcp, dstp, zero128, k):
    degagg = _agg_scalar(nm, srcp, dstp, zero128)
    deg = nm * (degagg + 1.0)
    dis = jax.lax.rsqrt(jnp.maximum(deg, 1.0)) * (deg > 0).astype(jnp.float32)
    d2n = dis * dis * nm
    hw = h @ W
    hwp = jnp.zeros((NPAD, H), jnp.float32).at[:N].set(hw * dis[:, None])
    agg = _agg_rows(hwp, srcp, dstp, zero128)
    h1 = jax.nn.relu((agg * dis[:, None] + hw * d2n[:, None] + b) * nm[:, None])
    hs = (h1 @ Ws)[:, 0]
    sagg = _agg_scalar(hs * dis, srcp, dstp, zero128)
    score = (sagg * dis + hs * d2n + bs[0]) * nm
    masked = jnp.where(nm > 0, score, -jnp.inf)
    _, idx = jax.lax.top_k(masked, k)
    new_nm = jnp.zeros((N,), jnp.float32).at[idx].set(1.0)
    h2 = h1 * jnp.tanh(score)[:, None] * new_nm[:, None]
    return h2, new_nm


def kernel(x, edge_index, batch, W1, b1, Ws1, bs1, W2, b2, Ws2, bs2, W3, b3, Ws3, bs3, Wc1, bc1, Wc2, bc2, Wc3, bc3):
    pad = jnp.full((EPAD - E,), NPAD - 1, jnp.int32)
    srcp = jnp.concatenate([edge_index[0], pad])
    dstp = jnp.concatenate([edge_index[1], pad])
    zero128 = jnp.zeros((RPT, 128), jnp.float32)
    nm = jnp.ones((N,), jnp.float32)

    h, nm = _layer(x, W1, b1, Ws1, bs1, nm, srcp, dstp, zero128, K1)
    h = _rbf(h, nm, K1)
    r1 = _readout(h, nm, K1)
    h, nm = _layer(h, W2, b2, Ws2, bs2, nm, srcp, dstp, zero128, K2)
    h = _rbf(h, nm, K2)
    r2 = _readout(h, nm, K2)
    h, nm = _layer(h, W3, b3, Ws3, bs3, nm, srcp, dstp, zero128, K3)
    h = _rbf(h, nm, K3)
    r3 = _readout(h, nm, K3)
    z = r1 + r2 + r3
    return _mlp_head(z, Wc1, bc1, Wc2, bc2, Wc3, bc3)
